# baseline (speedup 1.0000x reference)
import math

import jax
import jax.numpy as jnp
from jax import lax
from jax.experimental import pallas as pl
from jax.experimental.pallas import tpu as pltpu
from jax.experimental.pallas import tpu_sc as plsc

_N = 10000
_E = 320000
_D_IN = 128
_D_OUT = 16
_N_LABELS = 32
_S = 64.0
_M = 0.1
_COS_M = math.cos(_M)
_SIN_M = math.sin(_M)
_TH = math.cos(math.pi - _M)
_MM = math.sin(math.pi - _M) * _M

_NC = 2
_NS = 16
_NW = _NC * _NS
_EPW = _E // _NW
_C = 80
_K = _EPW // _C
_NBUF = 25
_G = _K // _NBUF
_RPS = _N // _NS
_NP = 10240
_HR = _NP // _NS
_HRS = _HR // _NS

_ROWS_TC = 2000


def _project(x, wl, wr, bl2, br2):
    grid = (_N // _ROWS_TC,)

    def body(x_ref, wl_ref, wr_ref, bl_ref, br_ref, y_ref, z_ref):
        xb = x_ref[...]
        dn = (((1,), (1,)), ((), ()))
        y_ref[...] = lax.dot_general(xb, wl_ref[...], dn,
                                     preferred_element_type=jnp.float32)
        z_ref[...] = (
            lax.dot_general(xb, wr_ref[...], dn,
                            preferred_element_type=jnp.float32)
            + bl_ref[...] + br_ref[...]
        )

    return pl.pallas_call(
        body,
        grid=grid,
        in_specs=[
            pl.BlockSpec((_ROWS_TC, _D_IN), lambda i: (i, 0)),
            pl.BlockSpec((_D_OUT, _D_IN), lambda i: (0, 0)),
            pl.BlockSpec((_D_OUT, _D_IN), lambda i: (0, 0)),
            pl.BlockSpec((1, _D_OUT), lambda i: (0, 0)),
            pl.BlockSpec((1, _D_OUT), lambda i: (0, 0)),
        ],
        out_specs=[
            pl.BlockSpec((_ROWS_TC, _D_OUT), lambda i: (i, 0)),
            pl.BlockSpec((_ROWS_TC, _D_OUT), lambda i: (i, 0)),
        ],
        out_shape=[
            jax.ShapeDtypeStruct((_N, _D_OUT), jnp.float32),
            jax.ShapeDtypeStruct((_N, _D_OUT), jnp.float32),
        ],
    )(x, wl, wr, bl2, br2)


def _hist_sc(ei):
    mesh = plsc.VectorSubcoreMesh(
        core_axis_name="c", subcore_axis_name="s",
        num_cores=_NC, num_subcores=_NS,
    )

    @pl.kernel(
        out_type=jax.ShapeDtypeStruct((_NC, _NP, _D_OUT), jnp.float32),
        mesh=mesh,
        scratch_types=[
            pltpu.VMEM((_K, _C), jnp.int32),
            pltpu.VMEM((_HR, _D_OUT), jnp.float32),
            pltpu.VMEM((_HR, _D_OUT), jnp.float32),
            pltpu.VMEM((_NS // 2, _HRS, _D_OUT), jnp.float32),
            pltpu.VMEM_SHARED((_NS // 2, _HR, _D_OUT), jnp.float32),
        ],
        compiler_params=pltpu.CompilerParams(use_tc_tiling_on_sc=False,
                                             needs_layout_passes=False),
    )
    def hist(ei_hbm, cnt_out, dst_v, hist_v, tmp_v, red_v, hist_s):
        cid = lax.axis_index("c")
        sid = lax.axis_index("s")
        w = cid * _NS + sid

        ones16 = jnp.ones((_D_OUT,), jnp.float32)
        zeros16 = jnp.zeros((_D_OUT,), jnp.float32)

        pltpu.sync_copy(ei_hbm.at[1].at[w], dst_v)

        def fill_zero(i, carry):
            hist_v[i, :] = zeros16
            return carry
        lax.fori_loop(0, _HR, fill_zero, 0)

        def hist_chunk(j, carry):
            for l in range(_C // _D_OUT):
                d = dst_v[j, pl.ds(l * _D_OUT, _D_OUT)]
                r = jnp.right_shift(d, 4)
                c = jnp.bitwise_and(d, 15)
                plsc.addupdate_scatter(hist_v, [r, c], ones16)
            return carry
        lax.fori_loop(0, _K, hist_chunk, 0)

        hrow0 = sid * _HRS
        half = _NS // 2

        @pl.when(sid < half)
        def _():
            pltpu.sync_copy(hist_v, hist_s.at[sid])
        plsc.subcore_barrier()
        for t in range(half):
            pltpu.sync_copy(hist_s.at[t].at[pl.ds(hrow0, _HRS)], red_v.at[t])

        def acc_round1(r, carry):
            acc = red_v[0, r, :]
            for t in range(1, half):
                acc = acc + red_v[t, r, :]
            tmp_v[r, :] = acc
            return carry
        lax.fori_loop(0, _HRS, acc_round1, 0)

        plsc.subcore_barrier()

        @pl.when(sid >= half)
        def _():
            pltpu.sync_copy(hist_v, hist_s.at[sid - half])
        plsc.subcore_barrier()
        for t in range(half):
            pltpu.sync_copy(hist_s.at[t].at[pl.ds(hrow0, _HRS)], red_v.at[t])

        def acc_round2(r, carry):
            acc = tmp_v[r, :]
            for t in range(half):
                acc = acc + red_v[t, r, :]
            tmp_v[r, :] = acc
            return carry
        lax.fori_loop(0, _HRS, acc_round2, 0)

        def expand(r, carry):
            for c in range(_D_OUT):
                val = plsc.load_gather(
                    tmp_v, [jnp.full((_D_OUT,), 0, jnp.int32) + r,
                            jnp.full((_D_OUT,), c, jnp.int32)])
                hist_v[r * _D_OUT + c, :] = val
            return carry
        lax.fori_loop(0, _HRS, expand, 0)
        pltpu.sync_copy(hist_v, cnt_out.at[cid].at[pl.ds(sid * _HR, _HR)])

    return hist(ei)


def _segment_sum_sc(y, ei):
    mesh = plsc.VectorSubcoreMesh(
        core_axis_name="c", subcore_axis_name="s",
        num_cores=_NC, num_subcores=_NS,
    )

    @pl.kernel(
        out_type=jax.ShapeDtypeStruct((_NC, _N, _D_OUT), jnp.float32),
        mesh=mesh,
        scratch_types=[
            pltpu.VMEM((_K, _C), jnp.int32),
            pltpu.VMEM((_K, _C), jnp.int32),
            pltpu.VMEM((2, _NBUF, _C, _D_OUT), jnp.float32),
            pltpu.VMEM((_RPS, _D_OUT), jnp.float32),
            pltpu.VMEM_SHARED((_N, _D_OUT), jnp.float32),
            pltpu.SemaphoreType.DMA,
            pltpu.SemaphoreType.DMA,
            pltpu.SemaphoreType.DMA,
        ],
        compiler_params=pltpu.CompilerParams(use_tc_tiling_on_sc=False,
                                             needs_layout_passes=False),
    )
    def seg(y_hbm, ei_hbm, agg_out,
            src_v, dst_v, rows_v, tmp_v, agg_s,
            gsem0, gsem1, ssem):
        cid = lax.axis_index("c")
        sid = lax.axis_index("s")
        w = cid * _NS + sid

        ones16 = jnp.ones((_D_OUT,), jnp.float32)
        zeros16 = jnp.zeros((_D_OUT,), jnp.float32)

        def fire_gathers(base, bank, sem):
            for i in range(_NBUF):
                pltpu.async_copy(y_hbm.at[src_v.at[base + i]],
                                 rows_v.at[bank].at[i], sem)

        def drain_gathers(bank, sem):
            for i in range(_NBUF):
                pltpu.make_async_copy(y_hbm.at[src_v.at[i]],
                                      rows_v.at[bank].at[i], sem).wait()

        def do_scatters(base, bank):
            descs = []
            for i in range(_NBUF):
                descs.append(
                    pltpu.async_copy(rows_v.at[bank].at[i],
                                     agg_s.at[dst_v.at[base + i]],
                                     ssem, add=True))
            for d in descs:
                d.wait()

        pltpu.sync_copy(ei_hbm.at[0].at[w], src_v)
        pltpu.sync_copy(ei_hbm.at[1].at[w], dst_v)

        fire_gathers(0, 0, gsem0)

        def fill_zero(i, carry):
            tmp_v[i, :] = zeros16
            return carry
        lax.fori_loop(0, _RPS, fill_zero, 0)

        row0 = sid * _RPS
        pltpu.sync_copy(tmp_v, agg_s.at[pl.ds(row0, _RPS)])

        plsc.subcore_barrier()

        def group(g, carry):
            @pl.when(g % 2 == 0)
            def _():
                drain_gathers(0, gsem0)

                @pl.when(g + 1 < _G)
                def _():
                    fire_gathers((g + 1) * _NBUF, 1, gsem1)
                do_scatters(g * _NBUF, 0)

            @pl.when(g % 2 == 1)
            def _():
                drain_gathers(1, gsem1)

                @pl.when(g + 1 < _G)
                def _():
                    fire_gathers((g + 1) * _NBUF, 0, gsem0)
                do_scatters(g * _NBUF, 1)
            return carry
        lax.fori_loop(0, _G, group, 0)

        pltpu.sync_copy(agg_s.at[pl.ds(row0, _RPS)], tmp_v)
        pltpu.sync_copy(tmp_v, agg_out.at[cid].at[pl.ds(row0, _RPS)])

    return seg(y, ei)


def _head(agg_p, cnt_p, z_p, label_p, weight):
    rows = 256
    grid = (pl.cdiv(_N // 8, rows),)
    _PK = 8
    _LW = _PK * _N_LABELS

    def body(a_ref, c_ref, z_ref, l_ref, w_ref, feat_ref, out_ref):
        aggs = a_ref[0] + a_ref[1]
        cnts = c_ref[0] + c_ref[1]
        mean = aggs / jnp.maximum(cnts, 1.0)
        h = mean + z_ref[...]
        feat_ref[...] = h
        hr = jnp.maximum(h, 0.0)

        r8 = lax.broadcasted_iota(jnp.int32, (_PK * _D_OUT, _PK), 0)
        c8 = lax.broadcasted_iota(jnp.int32, (_PK * _D_OUT, _PK), 1)
        s8 = jnp.where(r8 // _D_OUT == c8, 1.0, 0.0)
        sums = jnp.dot(hr * hr, s8, preferred_element_type=jnp.float32)
        inv = 1.0 / jnp.maximum(jnp.sqrt(sums), 1e-12)
        invb = jnp.dot(inv, s8.T, preferred_element_type=jnp.float32)
        xn = hr * invb

        wv = w_ref[...]
        wn = wv / jnp.maximum(
            jnp.sqrt(jnp.sum(wv * wv, axis=1, keepdims=True)), 1e-12)
        wt = jnp.tile(wn.T, (_PK, _PK))
        rw = lax.broadcasted_iota(jnp.int32, (_PK * _D_OUT, _LW), 0)
        cw = lax.broadcasted_iota(jnp.int32, (_PK * _D_OUT, _LW), 1)
        w8 = jnp.where(rw // _D_OUT == cw // _N_LABELS, wt, 0.0)
        cos = jnp.dot(xn, w8, preferred_element_type=jnp.float32)

        sine = jnp.sqrt(jnp.clip(1.0 - cos * cos, 0.0, 1.0))
        phi = cos * _COS_M - sine * _SIN_M
        phi = jnp.where(cos > _TH, phi, cos - _MM)

        cls = lax.broadcasted_iota(jnp.int32, (rows, _LW), 1) % _N_LABELS
        rs = lax.broadcasted_iota(jnp.int32, (_PK, _LW), 0)
        cs = lax.broadcasted_iota(jnp.int32, (_PK, _LW), 1)
        s32 = jnp.where(rs == cs // _N_LABELS, 1.0, 0.0)
        labelb = jnp.dot(l_ref[...], s32, preferred_element_type=jnp.float32)
        onehot = cls.astype(jnp.float32) == labelb
        out_ref[...] = jnp.where(onehot, phi, cos) * _S

    return pl.pallas_call(
        body,
        grid=grid,
        in_specs=[
            pl.BlockSpec((_NC, rows, _PK * _D_OUT), lambda i: (0, i, 0)),
            pl.BlockSpec((_NC, rows, _PK * _D_OUT), lambda i: (0, i, 0)),
            pl.BlockSpec((rows, _PK * _D_OUT), lambda i: (i, 0)),
            pl.BlockSpec((rows, _PK), lambda i: (i, 0)),
            pl.BlockSpec((_N_LABELS, _D_OUT), lambda i: (0, 0)),
        ],
        out_specs=[
            pl.BlockSpec((rows, _PK * _D_OUT), lambda i: (i, 0)),
            pl.BlockSpec((rows, _LW), lambda i: (i, 0)),
        ],
        out_shape=[
            jax.ShapeDtypeStruct((_N // _PK, _PK * _D_OUT), jnp.float32),
            jax.ShapeDtypeStruct((_N // _PK, _LW), jnp.float32),
        ],
    )(agg_p, cnt_p, z_p, label_p, weight)


def kernel(x, edge_index, label, Wl, bl, Wr, br, weight):
    ei = edge_index.reshape(2, _NW, _K, _C)
    cnt = _hist_sc(ei)
    y, z = _project(x, Wl, Wr, bl.reshape(1, _D_OUT), br.reshape(1, _D_OUT))
    agg = _segment_sum_sc(y, ei)
    np = _N // 8
    feat_p, out_p = _head(
        agg.reshape(_NC, np, 128), cnt.reshape(_NC, _NP // 8, 128),
        z.reshape(np, 128), label.astype(jnp.float32).reshape(np, 8), weight)
    return (feat_p.reshape(_N, _D_OUT), out_p.reshape(_N, _N_LABELS))


# speedup vs baseline: 1.0510x; 1.0510x over previous
import math

import jax
import jax.numpy as jnp
from jax import lax
from jax.experimental import pallas as pl
from jax.experimental.pallas import tpu as pltpu
from jax.experimental.pallas import tpu_sc as plsc

_N = 10000
_E = 320000
_D_IN = 128
_D_OUT = 16
_N_LABELS = 32
_S = 64.0
_M = 0.1
_COS_M = math.cos(_M)
_SIN_M = math.sin(_M)
_TH = math.cos(math.pi - _M)
_MM = math.sin(math.pi - _M) * _M

_NC = 2
_NS = 16
_NW = _NC * _NS
_EPW = _E // _NW
_C = 125
_K = _EPW // _C
_NBUF = 10
_G = _K // _NBUF
_RPS = _N // _NS
_NP = 10240
_HR = _NP // _NS
_HRS = _HR // _NS

_ROWS_TC = 2000


def _project(x, wl, wr, bl2, br2):
    grid = (_N // _ROWS_TC,)

    def body(x_ref, wl_ref, wr_ref, bl_ref, br_ref, y_ref, z_ref):
        xb = x_ref[...]
        dn = (((1,), (1,)), ((), ()))
        y_ref[...] = lax.dot_general(xb, wl_ref[...], dn,
                                     preferred_element_type=jnp.float32)
        z_ref[...] = (
            lax.dot_general(xb, wr_ref[...], dn,
                            preferred_element_type=jnp.float32)
            + bl_ref[...] + br_ref[...]
        )

    return pl.pallas_call(
        body,
        grid=grid,
        in_specs=[
            pl.BlockSpec((_ROWS_TC, _D_IN), lambda i: (i, 0)),
            pl.BlockSpec((_D_OUT, _D_IN), lambda i: (0, 0)),
            pl.BlockSpec((_D_OUT, _D_IN), lambda i: (0, 0)),
            pl.BlockSpec((1, _D_OUT), lambda i: (0, 0)),
            pl.BlockSpec((1, _D_OUT), lambda i: (0, 0)),
        ],
        out_specs=[
            pl.BlockSpec((_ROWS_TC, _D_OUT), lambda i: (i, 0)),
            pl.BlockSpec((_ROWS_TC, _D_OUT), lambda i: (i, 0)),
        ],
        out_shape=[
            jax.ShapeDtypeStruct((_N, _D_OUT), jnp.float32),
            jax.ShapeDtypeStruct((_N, _D_OUT), jnp.float32),
        ],
    )(x, wl, wr, bl2, br2)


def _segment_sum_sc(y, ei):
    mesh = plsc.VectorSubcoreMesh(
        core_axis_name="c", subcore_axis_name="s",
        num_cores=_NC, num_subcores=_NS,
    )

    @pl.kernel(
        out_type=[
            jax.ShapeDtypeStruct((_NC, _N, _D_OUT), jnp.float32),
            jax.ShapeDtypeStruct((_NC, _NP, _D_OUT), jnp.float32),
        ],
        mesh=mesh,
        scratch_types=[
            pltpu.VMEM((_K, _C), jnp.int32),
            pltpu.VMEM((_K, _C), jnp.int32),
            pltpu.VMEM((2, _NBUF, _C, _D_OUT), jnp.float32),
            pltpu.VMEM((_HR, _D_OUT), jnp.float32),
            pltpu.VMEM((_HR, _D_OUT), jnp.float32),
            pltpu.VMEM((_NS // 2, _HRS, _D_OUT), jnp.float32),
            pltpu.VMEM_SHARED((_N, _D_OUT), jnp.float32),
            pltpu.VMEM_SHARED((_NS // 2, _HR, _D_OUT), jnp.float32),
            pltpu.SemaphoreType.DMA,
            pltpu.SemaphoreType.DMA,
            pltpu.SemaphoreType.DMA,
        ],
        compiler_params=pltpu.CompilerParams(use_tc_tiling_on_sc=False,
                                             needs_layout_passes=False),
    )
    def seg(y_hbm, ei_hbm, agg_out, cnt_out,
            src_v, dst_v, rows_v, tmp_v, hist_v, red_v, agg_s, hist_s,
            gsem0, gsem1, ssem):
        cid = lax.axis_index("c")
        sid = lax.axis_index("s")
        w = cid * _NS + sid

        ones16 = jnp.ones((_D_OUT,), jnp.float32)
        zeros16 = jnp.zeros((_D_OUT,), jnp.float32)

        def fire_gathers(base, bank, sem):
            for i in range(_NBUF):
                pltpu.async_copy(y_hbm.at[src_v.at[base + i]],
                                 rows_v.at[bank].at[i], sem)

        def drain_gathers(bank, sem):
            for i in range(_NBUF):
                pltpu.make_async_copy(y_hbm.at[src_v.at[i]],
                                      rows_v.at[bank].at[i], sem).wait()

        def do_scatters(base, bank):
            descs = []
            for i in range(_NBUF):
                descs.append(
                    pltpu.async_copy(rows_v.at[bank].at[i],
                                     agg_s.at[dst_v.at[base + i]],
                                     ssem, add=True))
            for d in descs:
                d.wait()

        def hist_chunk(j, carry):
            for l in range(_C // _D_OUT):
                d = dst_v[j, pl.ds(l * _D_OUT, _D_OUT)]
                r = jnp.right_shift(d, 4)
                c = jnp.bitwise_and(d, 15)
                plsc.addupdate_scatter(hist_v, [r, c], ones16)
            return carry

        pltpu.sync_copy(ei_hbm.at[0].at[w], src_v)
        pltpu.sync_copy(ei_hbm.at[1].at[w], dst_v)

        fire_gathers(0, 0, gsem0)

        def fill_zero(i, carry):
            tmp_v[i, :] = zeros16
            hist_v[i, :] = zeros16
            return carry
        lax.fori_loop(0, _HR, fill_zero, 0)

        row0 = sid * _RPS
        pltpu.sync_copy(tmp_v.at[pl.ds(0, _RPS)], agg_s.at[pl.ds(row0, _RPS)])

        plsc.subcore_barrier()

        def group(g, carry):
            @pl.when(g % 2 == 0)
            def _():
                drain_gathers(0, gsem0)

                @pl.when(g + 1 < _G)
                def _():
                    fire_gathers((g + 1) * _NBUF, 1, gsem1)
                lax.fori_loop(g * _NBUF, (g + 1) * _NBUF, hist_chunk, 0)
                do_scatters(g * _NBUF, 0)

            @pl.when(g % 2 == 1)
            def _():
                drain_gathers(1, gsem1)

                @pl.when(g + 1 < _G)
                def _():
                    fire_gathers((g + 1) * _NBUF, 0, gsem0)
                lax.fori_loop(g * _NBUF, (g + 1) * _NBUF, hist_chunk, 0)
                do_scatters(g * _NBUF, 1)
            return carry
        lax.fori_loop(0, _G, group, 0)

        hrow0 = sid * _HRS
        half = _NS // 2

        @pl.when(sid < half)
        def _():
            pltpu.sync_copy(hist_v, hist_s.at[sid])
        plsc.subcore_barrier()
        for t in range(half):
            pltpu.sync_copy(hist_s.at[t].at[pl.ds(hrow0, _HRS)], red_v.at[t])

        def acc_round1(r, carry):
            acc = red_v[0, r, :]
            for t in range(1, half):
                acc = acc + red_v[t, r, :]
            tmp_v[r, :] = acc
            return carry
        lax.fori_loop(0, _HRS, acc_round1, 0)

        plsc.subcore_barrier()

        @pl.when(sid >= half)
        def _():
            pltpu.sync_copy(hist_v, hist_s.at[sid - half])
        plsc.subcore_barrier()
        for t in range(half):
            pltpu.sync_copy(hist_s.at[t].at[pl.ds(hrow0, _HRS)], red_v.at[t])

        def acc_round2(r, carry):
            acc = tmp_v[r, :]
            for t in range(half):
                acc = acc + red_v[t, r, :]
            tmp_v[r, :] = acc
            return carry
        lax.fori_loop(0, _HRS, acc_round2, 0)

        def expand(r, carry):
            for c in range(_D_OUT):
                val = plsc.load_gather(
                    tmp_v, [jnp.full((_D_OUT,), 0, jnp.int32) + r,
                            jnp.full((_D_OUT,), c, jnp.int32)])
                hist_v[r * _D_OUT + c, :] = val
            return carry
        lax.fori_loop(0, _HRS, expand, 0)
        pltpu.sync_copy(hist_v, cnt_out.at[cid].at[pl.ds(sid * _HR, _HR)])

        pltpu.sync_copy(agg_s.at[pl.ds(row0, _RPS)], tmp_v.at[pl.ds(0, _RPS)])
        pltpu.sync_copy(tmp_v.at[pl.ds(0, _RPS)],
                        agg_out.at[cid].at[pl.ds(row0, _RPS)])

    return seg(y, ei)


def _head(agg_p, cnt_p, z_p, label_p, weight):
    rows = 256
    grid = (pl.cdiv(_N // 8, rows),)
    _PK = 8
    _LW = _PK * _N_LABELS

    def body(a_ref, c_ref, z_ref, l_ref, w_ref, feat_ref, out_ref):
        aggs = a_ref[0] + a_ref[1]
        cnts = c_ref[0] + c_ref[1]
        mean = aggs / jnp.maximum(cnts, 1.0)
        h = mean + z_ref[...]
        feat_ref[...] = h
        hr = jnp.maximum(h, 0.0)

        r8 = lax.broadcasted_iota(jnp.int32, (_PK * _D_OUT, _PK), 0)
        c8 = lax.broadcasted_iota(jnp.int32, (_PK * _D_OUT, _PK), 1)
        s8 = jnp.where(r8 // _D_OUT == c8, 1.0, 0.0)
        sums = jnp.dot(hr * hr, s8, preferred_element_type=jnp.float32)
        inv = 1.0 / jnp.maximum(jnp.sqrt(sums), 1e-12)
        invb = jnp.dot(inv, s8.T, preferred_element_type=jnp.float32)
        xn = hr * invb

        wv = w_ref[...]
        wn = wv / jnp.maximum(
            jnp.sqrt(jnp.sum(wv * wv, axis=1, keepdims=True)), 1e-12)
        wt = jnp.tile(wn.T, (_PK, _PK))
        rw = lax.broadcasted_iota(jnp.int32, (_PK * _D_OUT, _LW), 0)
        cw = lax.broadcasted_iota(jnp.int32, (_PK * _D_OUT, _LW), 1)
        w8 = jnp.where(rw // _D_OUT == cw // _N_LABELS, wt, 0.0)
        cos = jnp.dot(xn, w8, preferred_element_type=jnp.float32)

        sine = jnp.sqrt(jnp.clip(1.0 - cos * cos, 0.0, 1.0))
        phi = cos * _COS_M - sine * _SIN_M
        phi = jnp.where(cos > _TH, phi, cos - _MM)

        cls = lax.broadcasted_iota(jnp.int32, (rows, _LW), 1) % _N_LABELS
        rs = lax.broadcasted_iota(jnp.int32, (_PK, _LW), 0)
        cs = lax.broadcasted_iota(jnp.int32, (_PK, _LW), 1)
        s32 = jnp.where(rs == cs // _N_LABELS, 1.0, 0.0)
        labelb = jnp.dot(l_ref[...], s32, preferred_element_type=jnp.float32)
        onehot = cls.astype(jnp.float32) == labelb
        out_ref[...] = jnp.where(onehot, phi, cos) * _S

    return pl.pallas_call(
        body,
        grid=grid,
        in_specs=[
            pl.BlockSpec((_NC, rows, _PK * _D_OUT), lambda i: (0, i, 0)),
            pl.BlockSpec((_NC, rows, _PK * _D_OUT), lambda i: (0, i, 0)),
            pl.BlockSpec((rows, _PK * _D_OUT), lambda i: (i, 0)),
            pl.BlockSpec((rows, _PK), lambda i: (i, 0)),
            pl.BlockSpec((_N_LABELS, _D_OUT), lambda i: (0, 0)),
        ],
        out_specs=[
            pl.BlockSpec((rows, _PK * _D_OUT), lambda i: (i, 0)),
            pl.BlockSpec((rows, _LW), lambda i: (i, 0)),
        ],
        out_shape=[
            jax.ShapeDtypeStruct((_N // _PK, _PK * _D_OUT), jnp.float32),
            jax.ShapeDtypeStruct((_N // _PK, _LW), jnp.float32),
        ],
    )(agg_p, cnt_p, z_p, label_p, weight)


def kernel(x, edge_index, label, Wl, bl, Wr, br, weight):
    y, z = _project(x, Wl, Wr, bl.reshape(1, _D_OUT), br.reshape(1, _D_OUT))
    ei = edge_index.reshape(2, _NW, _K, _C)
    agg, cnt = _segment_sum_sc(y, ei)
    np = _N // 8
    feat_p, out_p = _head(
        agg.reshape(_NC, np, 128), cnt.reshape(_NC, _NP // 8, 128),
        z.reshape(np, 128), label.astype(jnp.float32).reshape(np, 8), weight)
    return (feat_p.reshape(_N, _D_OUT), out_p.reshape(_N, _N_LABELS))


# speedup vs baseline: 1.0945x; 1.0413x over previous
import math

import jax
import jax.numpy as jnp
from jax import lax
from jax.experimental import pallas as pl
from jax.experimental.pallas import tpu as pltpu
from jax.experimental.pallas import tpu_sc as plsc

_N = 10000
_E = 320000
_D_IN = 128
_D_OUT = 16
_N_LABELS = 32
_S = 64.0
_M = 0.1
_COS_M = math.cos(_M)
_SIN_M = math.sin(_M)
_TH = math.cos(math.pi - _M)
_MM = math.sin(math.pi - _M) * _M

_NC = 2
_NS = 16
_NW = _NC * _NS
_EPW = _E // _NW
_C = 80
_K = _EPW // _C
_NBUF = 25
_G = _K // _NBUF
_RPS = _N // _NS
_NP = 10240
_HR = _NP // _NS
_HRS = _HR // _NS

_ROWS_TC = 1000


def _project(x, wl, wr, bl2, br2):
    grid = (_N // _ROWS_TC,)

    def body(x_ref, wl_ref, wr_ref, bl_ref, br_ref, y_ref, z_ref):
        xb = x_ref[...]
        dn = (((1,), (1,)), ((), ()))
        y_ref[...] = lax.dot_general(xb, wl_ref[...], dn,
                                     preferred_element_type=jnp.float32)
        z_ref[...] = (
            lax.dot_general(xb, wr_ref[...], dn,
                            preferred_element_type=jnp.float32)
            + bl_ref[...] + br_ref[...]
        )

    return pl.pallas_call(
        body,
        grid=grid,
        in_specs=[
            pl.BlockSpec((_ROWS_TC, _D_IN), lambda i: (i, 0)),
            pl.BlockSpec((_D_OUT, _D_IN), lambda i: (0, 0)),
            pl.BlockSpec((_D_OUT, _D_IN), lambda i: (0, 0)),
            pl.BlockSpec((1, _D_OUT), lambda i: (0, 0)),
            pl.BlockSpec((1, _D_OUT), lambda i: (0, 0)),
        ],
        out_specs=[
            pl.BlockSpec((_ROWS_TC, _D_OUT), lambda i: (i, 0)),
            pl.BlockSpec((_ROWS_TC, _D_OUT), lambda i: (i, 0)),
        ],
        out_shape=[
            jax.ShapeDtypeStruct((_N, _D_OUT), jnp.float32),
            jax.ShapeDtypeStruct((_N, _D_OUT), jnp.float32),
        ],
    )(x, wl, wr, bl2, br2)


def _segment_sum_sc(y, ei):
    mesh = plsc.VectorSubcoreMesh(
        core_axis_name="c", subcore_axis_name="s",
        num_cores=_NC, num_subcores=_NS,
    )

    @pl.kernel(
        out_type=[
            jax.ShapeDtypeStruct((_NC, _N, _D_OUT), jnp.float32),
            jax.ShapeDtypeStruct((_NC, _NP, _D_OUT), jnp.float32),
        ],
        mesh=mesh,
        scratch_types=[
            pltpu.VMEM((_K, _C), jnp.int32),
            pltpu.VMEM((_K, _C), jnp.int32),
            pltpu.VMEM((2, _NBUF, _C, _D_OUT), jnp.float32),
            pltpu.VMEM((_HR, _D_OUT), jnp.float32),
            pltpu.VMEM((_HR, _D_OUT), jnp.float32),
            pltpu.VMEM((_NS // 2, _HRS, _D_OUT), jnp.float32),
            pltpu.VMEM_SHARED((_N, _D_OUT), jnp.float32),
            pltpu.VMEM_SHARED((_NS // 2, _HR, _D_OUT), jnp.float32),
            pltpu.SemaphoreType.DMA,
            pltpu.SemaphoreType.DMA,
            pltpu.SemaphoreType.DMA,
        ],
        compiler_params=pltpu.CompilerParams(use_tc_tiling_on_sc=False,
                                             needs_layout_passes=False),
    )
    def seg(y_hbm, ei_hbm, agg_out, cnt_out,
            src_v, dst_v, rows_v, tmp_v, hist_v, red_v, agg_s, hist_s,
            gsem0, gsem1, ssem):
        cid = lax.axis_index("c")
        sid = lax.axis_index("s")
        w = cid * _NS + sid

        ones16 = jnp.ones((_D_OUT,), jnp.float32)
        zeros16 = jnp.zeros((_D_OUT,), jnp.float32)

        def fire_gathers(base, bank, sem):
            for i in range(_NBUF):
                pltpu.async_copy(y_hbm.at[src_v.at[base + i]],
                                 rows_v.at[bank].at[i], sem)

        def drain_gathers(bank, sem):
            for i in range(_NBUF):
                pltpu.make_async_copy(y_hbm.at[src_v.at[i]],
                                      rows_v.at[bank].at[i], sem).wait()

        def do_scatters(base, bank):
            descs = []
            for i in range(_NBUF):
                descs.append(
                    pltpu.async_copy(rows_v.at[bank].at[i],
                                     agg_s.at[dst_v.at[base + i]],
                                     ssem, add=True))
            for d in descs:
                d.wait()

        def hist_chunk(j, carry):
            for l in range(_C // _D_OUT):
                d = dst_v[j, pl.ds(l * _D_OUT, _D_OUT)]
                r = jnp.right_shift(d, 4)
                c = jnp.bitwise_and(d, 15)
                plsc.addupdate_scatter(hist_v, [r, c], ones16)
            return carry

        pltpu.sync_copy(ei_hbm.at[0].at[w], src_v)
        pltpu.sync_copy(ei_hbm.at[1].at[w], dst_v)

        fire_gathers(0, 0, gsem0)

        def fill_zero(i, carry):
            tmp_v[i, :] = zeros16
            hist_v[i, :] = zeros16
            return carry
        lax.fori_loop(0, _HR, fill_zero, 0)

        row0 = sid * _RPS
        pltpu.sync_copy(tmp_v.at[pl.ds(0, _RPS)], agg_s.at[pl.ds(row0, _RPS)])

        plsc.subcore_barrier()

        def group(g, carry):
            @pl.when(g % 2 == 0)
            def _():
                drain_gathers(0, gsem0)

                @pl.when(g + 1 < _G)
                def _():
                    fire_gathers((g + 1) * _NBUF, 1, gsem1)
                lax.fori_loop(g * _NBUF, (g + 1) * _NBUF, hist_chunk, 0)
                do_scatters(g * _NBUF, 0)

            @pl.when(g % 2 == 1)
            def _():
                drain_gathers(1, gsem1)

                @pl.when(g + 1 < _G)
                def _():
                    fire_gathers((g + 1) * _NBUF, 0, gsem0)
                lax.fori_loop(g * _NBUF, (g + 1) * _NBUF, hist_chunk, 0)
                do_scatters(g * _NBUF, 1)
            return carry
        lax.fori_loop(0, _G, group, 0)

        hrow0 = sid * _HRS
        half = _NS // 2

        @pl.when(sid < half)
        def _():
            pltpu.sync_copy(hist_v, hist_s.at[sid])
        plsc.subcore_barrier()
        for t in range(half):
            pltpu.sync_copy(hist_s.at[t].at[pl.ds(hrow0, _HRS)], red_v.at[t])

        def acc_round1(r, carry):
            acc = red_v[0, r, :]
            for t in range(1, half):
                acc = acc + red_v[t, r, :]
            tmp_v[r, :] = acc
            return carry
        lax.fori_loop(0, _HRS, acc_round1, 0)

        plsc.subcore_barrier()

        @pl.when(sid >= half)
        def _():
            pltpu.sync_copy(hist_v, hist_s.at[sid - half])
        plsc.subcore_barrier()
        for t in range(half):
            pltpu.sync_copy(hist_s.at[t].at[pl.ds(hrow0, _HRS)], red_v.at[t])

        def acc_round2(r, carry):
            acc = tmp_v[r, :]
            for t in range(half):
                acc = acc + red_v[t, r, :]
            tmp_v[r, :] = acc
            return carry
        lax.fori_loop(0, _HRS, acc_round2, 0)

        def expand(r, carry):
            for c in range(_D_OUT):
                val = plsc.load_gather(
                    tmp_v, [jnp.full((_D_OUT,), 0, jnp.int32) + r,
                            jnp.full((_D_OUT,), c, jnp.int32)])
                hist_v[r * _D_OUT + c, :] = val
            return carry
        lax.fori_loop(0, _HRS, expand, 0)
        pltpu.sync_copy(hist_v, cnt_out.at[cid].at[pl.ds(sid * _HR, _HR)])

        pltpu.sync_copy(agg_s.at[pl.ds(row0, _RPS)], tmp_v.at[pl.ds(0, _RPS)])
        pltpu.sync_copy(tmp_v.at[pl.ds(0, _RPS)],
                        agg_out.at[cid].at[pl.ds(row0, _RPS)])

    return seg(y, ei)


def _head(agg_p, cnt_p, z_p, label_p, weight):
    rows = 256
    grid = (pl.cdiv(_N // 8, rows),)
    _PK = 8
    _LW = _PK * _N_LABELS

    def body(a_ref, c_ref, z_ref, l_ref, w_ref, feat_ref, out_ref):
        aggs = a_ref[0] + a_ref[1]
        cnts = c_ref[0] + c_ref[1]
        mean = aggs / jnp.maximum(cnts, 1.0)
        h = mean + z_ref[...]
        feat_ref[...] = h
        hr = jnp.maximum(h, 0.0)

        r8 = lax.broadcasted_iota(jnp.int32, (_PK * _D_OUT, _PK), 0)
        c8 = lax.broadcasted_iota(jnp.int32, (_PK * _D_OUT, _PK), 1)
        s8 = jnp.where(r8 // _D_OUT == c8, 1.0, 0.0)
        sums = jnp.dot(hr * hr, s8, preferred_element_type=jnp.float32)
        inv = 1.0 / jnp.maximum(jnp.sqrt(sums), 1e-12)
        invb = jnp.dot(inv, s8.T, preferred_element_type=jnp.float32)
        xn = hr * invb

        wv = w_ref[...]
        wn = wv / jnp.maximum(
            jnp.sqrt(jnp.sum(wv * wv, axis=1, keepdims=True)), 1e-12)
        wt = jnp.tile(wn.T, (_PK, _PK))
        rw = lax.broadcasted_iota(jnp.int32, (_PK * _D_OUT, _LW), 0)
        cw = lax.broadcasted_iota(jnp.int32, (_PK * _D_OUT, _LW), 1)
        w8 = jnp.where(rw // _D_OUT == cw // _N_LABELS, wt, 0.0)
        cos = jnp.dot(xn, w8, preferred_element_type=jnp.float32)

        sine = jnp.sqrt(jnp.clip(1.0 - cos * cos, 0.0, 1.0))
        phi = cos * _COS_M - sine * _SIN_M
        phi = jnp.where(cos > _TH, phi, cos - _MM)

        cls = lax.broadcasted_iota(jnp.int32, (rows, _LW), 1) % _N_LABELS
        rs = lax.broadcasted_iota(jnp.int32, (_PK, _LW), 0)
        cs = lax.broadcasted_iota(jnp.int32, (_PK, _LW), 1)
        s32 = jnp.where(rs == cs // _N_LABELS, 1.0, 0.0)
        labelb = jnp.dot(l_ref[...], s32, preferred_element_type=jnp.float32)
        onehot = cls.astype(jnp.float32) == labelb
        out_ref[...] = jnp.where(onehot, phi, cos) * _S

    return pl.pallas_call(
        body,
        grid=grid,
        in_specs=[
            pl.BlockSpec((_NC, rows, _PK * _D_OUT), lambda i: (0, i, 0)),
            pl.BlockSpec((_NC, rows, _PK * _D_OUT), lambda i: (0, i, 0)),
            pl.BlockSpec((rows, _PK * _D_OUT), lambda i: (i, 0)),
            pl.BlockSpec((rows, _PK), lambda i: (i, 0)),
            pl.BlockSpec((_N_LABELS, _D_OUT), lambda i: (0, 0)),
        ],
        out_specs=[
            pl.BlockSpec((rows, _PK * _D_OUT), lambda i: (i, 0)),
            pl.BlockSpec((rows, _LW), lambda i: (i, 0)),
        ],
        out_shape=[
            jax.ShapeDtypeStruct((_N // _PK, _PK * _D_OUT), jnp.float32),
            jax.ShapeDtypeStruct((_N // _PK, _LW), jnp.float32),
        ],
    )(agg_p, cnt_p, z_p, label_p, weight)


def kernel(x, edge_index, label, Wl, bl, Wr, br, weight):
    y, z = _project(x, Wl, Wr, bl.reshape(1, _D_OUT), br.reshape(1, _D_OUT))
    ei = edge_index.reshape(2, _NW, _K, _C)
    agg, cnt = _segment_sum_sc(y, ei)
    np = _N // 8
    feat_p, out_p = _head(
        agg.reshape(_NC, np, 128), cnt.reshape(_NC, _NP // 8, 128),
        z.reshape(np, 128), label.astype(jnp.float32).reshape(np, 8), weight)
    return (feat_p.reshape(_N, _D_OUT), out_p.reshape(_N, _N_LABELS))


# speedup vs baseline: 1.1326x; 1.0348x over previous
import math

import jax
import jax.numpy as jnp
from jax import lax
from jax.experimental import pallas as pl
from jax.experimental.pallas import tpu as pltpu
from jax.experimental.pallas import tpu_sc as plsc

_N = 10000
_E = 320000
_D_IN = 128
_D_OUT = 16
_N_LABELS = 32
_S = 64.0
_M = 0.1
_COS_M = math.cos(_M)
_SIN_M = math.sin(_M)
_TH = math.cos(math.pi - _M)
_MM = math.sin(math.pi - _M) * _M

_NC = 2
_NS = 16
_NW = _NC * _NS
_EPW = _E // _NW
_C = 80
_K = _EPW // _C
_NBUF = 25
_G = _K // _NBUF
_RPS = _N // _NS
_NP = 10240
_HR = _NP // _NS
_HRS = _HR // _NS

_ROWS_TC = 2000


def _project(x, wl, wr, bl2, br2):
    grid = (_N // _ROWS_TC,)

    def body(x_ref, wl_ref, wr_ref, bl_ref, br_ref, y_ref, z_ref):
        xb = x_ref[...]
        dn = (((1,), (1,)), ((), ()))
        y_ref[...] = lax.dot_general(xb, wl_ref[...], dn,
                                     preferred_element_type=jnp.float32)
        z_ref[...] = (
            lax.dot_general(xb, wr_ref[...], dn,
                            preferred_element_type=jnp.float32)
            + bl_ref[...] + br_ref[...]
        )

    return pl.pallas_call(
        body,
        grid=grid,
        in_specs=[
            pl.BlockSpec((_ROWS_TC, _D_IN), lambda i: (i, 0)),
            pl.BlockSpec((_D_OUT, _D_IN), lambda i: (0, 0)),
            pl.BlockSpec((_D_OUT, _D_IN), lambda i: (0, 0)),
            pl.BlockSpec((1, _D_OUT), lambda i: (0, 0)),
            pl.BlockSpec((1, _D_OUT), lambda i: (0, 0)),
        ],
        out_specs=[
            pl.BlockSpec((_ROWS_TC, _D_OUT), lambda i: (i, 0)),
            pl.BlockSpec((_ROWS_TC, _D_OUT), lambda i: (i, 0)),
        ],
        out_shape=[
            jax.ShapeDtypeStruct((_N, _D_OUT), jnp.float32),
            jax.ShapeDtypeStruct((_N, _D_OUT), jnp.float32),
        ],
    )(x, wl, wr, bl2, br2)


def _segment_sum_sc(y, ei):
    mesh = plsc.VectorSubcoreMesh(
        core_axis_name="c", subcore_axis_name="s",
        num_cores=_NC, num_subcores=_NS,
    )

    @pl.kernel(
        out_type=[
            jax.ShapeDtypeStruct((_NC, _N, _D_OUT), jnp.float32),
            jax.ShapeDtypeStruct((_NC, _NP, _D_OUT), jnp.float32),
        ],
        mesh=mesh,
        scratch_types=[
            pltpu.VMEM((_K, _C), jnp.int32),
            pltpu.VMEM((_K, _C), jnp.int32),
            pltpu.VMEM((2, _NBUF, _C, _D_OUT), jnp.float32),
            pltpu.VMEM((_HR, _D_OUT), jnp.float32),
            pltpu.VMEM((_HR, _D_OUT), jnp.float32),
            pltpu.VMEM((_NS // 2, _HRS, _D_OUT), jnp.float32),
            pltpu.VMEM_SHARED((_N, _D_OUT), jnp.float32),
            pltpu.VMEM_SHARED((_NS // 2, _HR, _D_OUT), jnp.float32),
            pltpu.SemaphoreType.DMA,
            pltpu.SemaphoreType.DMA,
            pltpu.SemaphoreType.DMA,
        ],
        compiler_params=pltpu.CompilerParams(use_tc_tiling_on_sc=False,
                                             needs_layout_passes=False),
    )
    def seg(y_hbm, ei_hbm, agg_out, cnt_out,
            src_v, dst_v, rows_v, tmp_v, hist_v, red_v, agg_s, hist_s,
            gsem0, gsem1, ssem):
        cid = lax.axis_index("c")
        sid = lax.axis_index("s")
        w = cid * _NS + sid

        ones16 = jnp.ones((_D_OUT,), jnp.float32)
        zeros16 = jnp.zeros((_D_OUT,), jnp.float32)

        def fire_gathers(base, bank, sem):
            for i in range(_NBUF):
                pltpu.async_copy(y_hbm.at[src_v.at[base + i]],
                                 rows_v.at[bank].at[i], sem)

        def drain_gathers(bank, sem):
            for i in range(_NBUF):
                pltpu.make_async_copy(y_hbm.at[src_v.at[i]],
                                      rows_v.at[bank].at[i], sem).wait()

        def do_scatters(base, bank):
            descs = []
            for i in range(_NBUF):
                descs.append(
                    pltpu.async_copy(rows_v.at[bank].at[i],
                                     agg_s.at[dst_v.at[base + i]],
                                     ssem, add=True))
            for d in descs:
                d.wait()

        def hist_chunk(j, carry):
            for l in range(_C // _D_OUT):
                d = dst_v[j, pl.ds(l * _D_OUT, _D_OUT)]
                r = jnp.right_shift(d, 4)
                c = jnp.bitwise_and(d, 15)
                plsc.addupdate_scatter(hist_v, [r, c], ones16)
            return carry

        pltpu.sync_copy(ei_hbm.at[0].at[w], src_v)
        pltpu.sync_copy(ei_hbm.at[1].at[w], dst_v)

        fire_gathers(0, 0, gsem0)

        def fill_zero(i, carry):
            tmp_v[i, :] = zeros16
            hist_v[i, :] = zeros16
            return carry
        lax.fori_loop(0, _HR, fill_zero, 0)

        row0 = sid * _RPS
        pltpu.sync_copy(tmp_v.at[pl.ds(0, _RPS)], agg_s.at[pl.ds(row0, _RPS)])

        plsc.subcore_barrier()

        def group(g, carry):
            @pl.when(g % 2 == 0)
            def _():
                drain_gathers(0, gsem0)

                @pl.when(g + 1 < _G)
                def _():
                    fire_gathers((g + 1) * _NBUF, 1, gsem1)
                lax.fori_loop(g * _NBUF, (g + 1) * _NBUF, hist_chunk, 0)
                do_scatters(g * _NBUF, 0)

            @pl.when(g % 2 == 1)
            def _():
                drain_gathers(1, gsem1)

                @pl.when(g + 1 < _G)
                def _():
                    fire_gathers((g + 1) * _NBUF, 0, gsem0)
                lax.fori_loop(g * _NBUF, (g + 1) * _NBUF, hist_chunk, 0)
                do_scatters(g * _NBUF, 1)
            return carry
        lax.fori_loop(0, _G, group, 0)

        hrow0 = sid * _HRS
        half = _NS // 2

        @pl.when(sid < half)
        def _():
            pltpu.sync_copy(hist_v, hist_s.at[sid])
        plsc.subcore_barrier()
        for t in range(half):
            pltpu.sync_copy(hist_s.at[t].at[pl.ds(hrow0, _HRS)], red_v.at[t])

        def acc_round1(r, carry):
            acc = red_v[0, r, :]
            for t in range(1, half):
                acc = acc + red_v[t, r, :]
            tmp_v[r, :] = acc
            return carry
        lax.fori_loop(0, _HRS, acc_round1, 0)

        plsc.subcore_barrier()

        @pl.when(sid >= half)
        def _():
            pltpu.sync_copy(hist_v, hist_s.at[sid - half])
        plsc.subcore_barrier()
        for t in range(half):
            pltpu.sync_copy(hist_s.at[t].at[pl.ds(hrow0, _HRS)], red_v.at[t])

        def acc_round2(r, carry):
            acc = tmp_v[r, :]
            for t in range(half):
                acc = acc + red_v[t, r, :]
            tmp_v[r, :] = acc
            return carry
        lax.fori_loop(0, _HRS, acc_round2, 0)

        def expand(r, carry):
            for c in range(_D_OUT):
                val = plsc.load_gather(
                    tmp_v, [jnp.full((_D_OUT,), 0, jnp.int32) + r,
                            jnp.full((_D_OUT,), c, jnp.int32)])
                hist_v[r * _D_OUT + c, :] = val
            return carry
        lax.fori_loop(0, _HRS, expand, 0)
        pltpu.sync_copy(hist_v, cnt_out.at[cid].at[pl.ds(sid * _HR, _HR)])

        pltpu.sync_copy(agg_s.at[pl.ds(row0, _RPS)], tmp_v.at[pl.ds(0, _RPS)])
        pltpu.sync_copy(tmp_v.at[pl.ds(0, _RPS)],
                        agg_out.at[cid].at[pl.ds(row0, _RPS)])

    return seg(y, ei)


def _head(agg_p, cnt_p, z_p, label_p, weight):
    rows = 256
    grid = (pl.cdiv(_N // 8, rows),)
    _PK = 8
    _LW = _PK * _N_LABELS

    def body(a_ref, c_ref, z_ref, l_ref, w_ref, feat_ref, out_ref):
        aggs = a_ref[0] + a_ref[1]
        cnts = c_ref[0] + c_ref[1]
        mean = aggs / jnp.maximum(cnts, 1.0)
        h = mean + z_ref[...]
        feat_ref[...] = h
        hr = jnp.maximum(h, 0.0)

        r8 = lax.broadcasted_iota(jnp.int32, (_PK * _D_OUT, _PK), 0)
        c8 = lax.broadcasted_iota(jnp.int32, (_PK * _D_OUT, _PK), 1)
        s8 = jnp.where(r8 // _D_OUT == c8, 1.0, 0.0)
        sums = jnp.dot(hr * hr, s8, preferred_element_type=jnp.float32)
        inv = 1.0 / jnp.maximum(jnp.sqrt(sums), 1e-12)
        invb = jnp.dot(inv, s8.T, preferred_element_type=jnp.float32)
        xn = hr * invb

        wv = w_ref[...]
        wn = wv / jnp.maximum(
            jnp.sqrt(jnp.sum(wv * wv, axis=1, keepdims=True)), 1e-12)
        wt = jnp.tile(wn.T, (_PK, _PK))
        rw = lax.broadcasted_iota(jnp.int32, (_PK * _D_OUT, _LW), 0)
        cw = lax.broadcasted_iota(jnp.int32, (_PK * _D_OUT, _LW), 1)
        w8 = jnp.where(rw // _D_OUT == cw // _N_LABELS, wt, 0.0)
        cos = jnp.dot(xn, w8, preferred_element_type=jnp.float32)

        sine = jnp.sqrt(jnp.clip(1.0 - cos * cos, 0.0, 1.0))
        phi = cos * _COS_M - sine * _SIN_M
        phi = jnp.where(cos > _TH, phi, cos - _MM)

        cls = lax.broadcasted_iota(jnp.int32, (rows, _LW), 1) % _N_LABELS
        rs = lax.broadcasted_iota(jnp.int32, (_PK, _LW), 0)
        cs = lax.broadcasted_iota(jnp.int32, (_PK, _LW), 1)
        s32 = jnp.where(rs == cs // _N_LABELS, 1.0, 0.0)
        labelb = jnp.dot(l_ref[...], s32, preferred_element_type=jnp.float32)
        onehot = cls.astype(jnp.float32) == labelb
        out_ref[...] = jnp.where(onehot, phi, cos) * _S

    return pl.pallas_call(
        body,
        grid=grid,
        in_specs=[
            pl.BlockSpec((_NC, rows, _PK * _D_OUT), lambda i: (0, i, 0)),
            pl.BlockSpec((_NC, rows, _PK * _D_OUT), lambda i: (0, i, 0)),
            pl.BlockSpec((rows, _PK * _D_OUT), lambda i: (i, 0)),
            pl.BlockSpec((rows, _PK), lambda i: (i, 0)),
            pl.BlockSpec((_N_LABELS, _D_OUT), lambda i: (0, 0)),
        ],
        out_specs=[
            pl.BlockSpec((rows, _PK * _D_OUT), lambda i: (i, 0)),
            pl.BlockSpec((rows, _LW), lambda i: (i, 0)),
        ],
        out_shape=[
            jax.ShapeDtypeStruct((_N // _PK, _PK * _D_OUT), jnp.float32),
            jax.ShapeDtypeStruct((_N // _PK, _LW), jnp.float32),
        ],
    )(agg_p, cnt_p, z_p, label_p, weight)


def kernel(x, edge_index, label, Wl, bl, Wr, br, weight):
    y, z = _project(x, Wl, Wr, bl.reshape(1, _D_OUT), br.reshape(1, _D_OUT))
    ei = edge_index.reshape(2, _NW, _K, _C)
    agg, cnt = _segment_sum_sc(y, ei)
    np = _N // 8
    feat_p, out_p = _head(
        agg.reshape(_NC, np, 128), cnt.reshape(_NC, _NP // 8, 128),
        z.reshape(np, 128), label.astype(jnp.float32).reshape(np, 8), weight)
    return (feat_p.reshape(_N, _D_OUT), out_p.reshape(_N, _N_LABELS))


# speedup vs baseline: 1.1420x; 1.0083x over previous
import math

import jax
import jax.numpy as jnp
from jax import lax
from jax.experimental import pallas as pl
from jax.experimental.pallas import tpu as pltpu
from jax.experimental.pallas import tpu_sc as plsc

_N = 10000
_E = 320000
_D_IN = 128
_D_OUT = 16
_N_LABELS = 32
_S = 64.0
_M = 0.1
_COS_M = math.cos(_M)
_SIN_M = math.sin(_M)
_TH = math.cos(math.pi - _M)
_MM = math.sin(math.pi - _M) * _M

_NC = 2
_NS = 16
_NW = _NC * _NS
_EPW = _E // _NW
_C = 80
_K = _EPW // _C
_NBUF = 25
_G = _K // _NBUF
_RPS = _N // _NS
_NP = 10240
_HR = _NP // _NS
_HRS = _HR // _NS

_ROWS_TC = 2000


def _project(x, wl, wr, bl2, br2):
    grid = (_N // _ROWS_TC,)

    def body(x_ref, wl_ref, wr_ref, bl_ref, br_ref, y_ref, z_ref):
        xb = x_ref[...]
        dn = (((1,), (1,)), ((), ()))
        y_ref[...] = lax.dot_general(xb, wl_ref[...], dn,
                                     preferred_element_type=jnp.float32)
        z_ref[...] = (
            lax.dot_general(xb, wr_ref[...], dn,
                            preferred_element_type=jnp.float32)
            + bl_ref[...] + br_ref[...]
        )

    return pl.pallas_call(
        body,
        grid=grid,
        in_specs=[
            pl.BlockSpec((_ROWS_TC, _D_IN), lambda i: (i, 0)),
            pl.BlockSpec((_D_OUT, _D_IN), lambda i: (0, 0)),
            pl.BlockSpec((_D_OUT, _D_IN), lambda i: (0, 0)),
            pl.BlockSpec((1, _D_OUT), lambda i: (0, 0)),
            pl.BlockSpec((1, _D_OUT), lambda i: (0, 0)),
        ],
        out_specs=[
            pl.BlockSpec((_ROWS_TC, _D_OUT), lambda i: (i, 0)),
            pl.BlockSpec((_ROWS_TC, _D_OUT), lambda i: (i, 0)),
        ],
        out_shape=[
            jax.ShapeDtypeStruct((_N, _D_OUT), jnp.float32),
            jax.ShapeDtypeStruct((_N, _D_OUT), jnp.float32),
        ],
    )(x, wl, wr, bl2, br2)


def _segment_sum_sc(y, ei):
    mesh = plsc.VectorSubcoreMesh(
        core_axis_name="c", subcore_axis_name="s",
        num_cores=_NC, num_subcores=_NS,
    )

    @pl.kernel(
        out_type=[
            jax.ShapeDtypeStruct((_NC, _N, _D_OUT), jnp.float32),
            jax.ShapeDtypeStruct((_NC, _NP, _D_OUT), jnp.float32),
        ],
        mesh=mesh,
        scratch_types=[
            pltpu.VMEM((_K, _C), jnp.int32),
            pltpu.VMEM((_K, _C), jnp.int32),
            pltpu.VMEM((2, _NBUF, _C, _D_OUT), jnp.float32),
            pltpu.VMEM((_HR, _D_OUT), jnp.float32),
            pltpu.VMEM((_HR, _D_OUT), jnp.float32),
            pltpu.VMEM((_NS // 2, _HRS, _D_OUT), jnp.float32),
            pltpu.VMEM_SHARED((_N, _D_OUT), jnp.float32),
            pltpu.VMEM_SHARED((_NS // 2, _HR, _D_OUT), jnp.float32),
            pltpu.SemaphoreType.DMA,
            pltpu.SemaphoreType.DMA,
            pltpu.SemaphoreType.DMA,
        ],
        compiler_params=pltpu.CompilerParams(use_tc_tiling_on_sc=False,
                                             needs_layout_passes=False),
    )
    def seg(y_hbm, ei_hbm, agg_out, cnt_out,
            src_v, dst_v, rows_v, tmp_v, hist_v, red_v, agg_s, hist_s,
            gsem0, gsem1, ssem):
        cid = lax.axis_index("c")
        sid = lax.axis_index("s")
        w = cid * _NS + sid

        ones16 = jnp.ones((_D_OUT,), jnp.float32)
        zeros16 = jnp.zeros((_D_OUT,), jnp.float32)

        def fire_gathers(base, bank, sem):
            for i in range(_NBUF):
                pltpu.async_copy(y_hbm.at[src_v.at[base + i]],
                                 rows_v.at[bank].at[i], sem)

        def drain_gathers(bank, sem):
            for i in range(_NBUF):
                pltpu.make_async_copy(y_hbm.at[src_v.at[i]],
                                      rows_v.at[bank].at[i], sem).wait()

        def fire_scatters(base, bank):
            descs = []
            for i in range(_NBUF):
                descs.append(
                    pltpu.async_copy(rows_v.at[bank].at[i],
                                     agg_s.at[dst_v.at[base + i]],
                                     ssem, add=True))
            return descs

        def hist_chunk(j, carry):
            for l in range(_C // _D_OUT):
                d = dst_v[j, pl.ds(l * _D_OUT, _D_OUT)]
                r = jnp.right_shift(d, 4)
                c = jnp.bitwise_and(d, 15)
                plsc.addupdate_scatter(hist_v, [r, c], ones16)
            return carry

        pltpu.sync_copy(ei_hbm.at[0].at[w], src_v)
        pltpu.sync_copy(ei_hbm.at[1].at[w], dst_v)

        fire_gathers(0, 0, gsem0)

        def fill_zero(i, carry):
            tmp_v[i, :] = zeros16
            hist_v[i, :] = zeros16
            return carry
        lax.fori_loop(0, _HR, fill_zero, 0)

        row0 = sid * _RPS
        pltpu.sync_copy(tmp_v.at[pl.ds(0, _RPS)], agg_s.at[pl.ds(row0, _RPS)])

        plsc.subcore_barrier()

        def group(g, carry):
            @pl.when(g % 2 == 0)
            def _():
                drain_gathers(0, gsem0)

                @pl.when(g + 1 < _G)
                def _():
                    fire_gathers((g + 1) * _NBUF, 1, gsem1)
                descs = fire_scatters(g * _NBUF, 0)
                lax.fori_loop(g * _NBUF, (g + 1) * _NBUF, hist_chunk, 0)
                for d in descs:
                    d.wait()

            @pl.when(g % 2 == 1)
            def _():
                drain_gathers(1, gsem1)

                @pl.when(g + 1 < _G)
                def _():
                    fire_gathers((g + 1) * _NBUF, 0, gsem0)
                descs = fire_scatters(g * _NBUF, 1)
                lax.fori_loop(g * _NBUF, (g + 1) * _NBUF, hist_chunk, 0)
                for d in descs:
                    d.wait()
            return carry
        lax.fori_loop(0, _G, group, 0)

        hrow0 = sid * _HRS
        half = _NS // 2

        @pl.when(sid < half)
        def _():
            pltpu.sync_copy(hist_v, hist_s.at[sid])
        plsc.subcore_barrier()
        for t in range(half):
            pltpu.sync_copy(hist_s.at[t].at[pl.ds(hrow0, _HRS)], red_v.at[t])

        def acc_round1(r, carry):
            acc = red_v[0, r, :]
            for t in range(1, half):
                acc = acc + red_v[t, r, :]
            tmp_v[r, :] = acc
            return carry
        lax.fori_loop(0, _HRS, acc_round1, 0)

        plsc.subcore_barrier()

        @pl.when(sid >= half)
        def _():
            pltpu.sync_copy(hist_v, hist_s.at[sid - half])
        plsc.subcore_barrier()
        for t in range(half):
            pltpu.sync_copy(hist_s.at[t].at[pl.ds(hrow0, _HRS)], red_v.at[t])

        def acc_round2(r, carry):
            acc = tmp_v[r, :]
            for t in range(half):
                acc = acc + red_v[t, r, :]
            tmp_v[r, :] = acc
            return carry
        lax.fori_loop(0, _HRS, acc_round2, 0)

        def expand(r, carry):
            for c in range(_D_OUT):
                val = plsc.load_gather(
                    tmp_v, [jnp.full((_D_OUT,), 0, jnp.int32) + r,
                            jnp.full((_D_OUT,), c, jnp.int32)])
                hist_v[r * _D_OUT + c, :] = val
            return carry
        lax.fori_loop(0, _HRS, expand, 0)
        pltpu.sync_copy(hist_v, cnt_out.at[cid].at[pl.ds(sid * _HR, _HR)])

        pltpu.sync_copy(agg_s.at[pl.ds(row0, _RPS)], tmp_v.at[pl.ds(0, _RPS)])
        pltpu.sync_copy(tmp_v.at[pl.ds(0, _RPS)],
                        agg_out.at[cid].at[pl.ds(row0, _RPS)])

    return seg(y, ei)


def _head(agg_p, cnt_p, z_p, label_p, weight):
    rows = 256
    grid = (pl.cdiv(_N // 8, rows),)
    _PK = 8
    _LW = _PK * _N_LABELS

    def body(a_ref, c_ref, z_ref, l_ref, w_ref, feat_ref, out_ref):
        aggs = a_ref[0] + a_ref[1]
        cnts = c_ref[0] + c_ref[1]
        mean = aggs / jnp.maximum(cnts, 1.0)
        h = mean + z_ref[...]
        feat_ref[...] = h
        hr = jnp.maximum(h, 0.0)

        r8 = lax.broadcasted_iota(jnp.int32, (_PK * _D_OUT, _PK), 0)
        c8 = lax.broadcasted_iota(jnp.int32, (_PK * _D_OUT, _PK), 1)
        s8 = jnp.where(r8 // _D_OUT == c8, 1.0, 0.0)
        sums = jnp.dot(hr * hr, s8, preferred_element_type=jnp.float32)
        inv = 1.0 / jnp.maximum(jnp.sqrt(sums), 1e-12)
        invb = jnp.dot(inv, s8.T, preferred_element_type=jnp.float32)
        xn = hr * invb

        wv = w_ref[...]
        wn = wv / jnp.maximum(
            jnp.sqrt(jnp.sum(wv * wv, axis=1, keepdims=True)), 1e-12)
        wt = jnp.tile(wn.T, (_PK, _PK))
        rw = lax.broadcasted_iota(jnp.int32, (_PK * _D_OUT, _LW), 0)
        cw = lax.broadcasted_iota(jnp.int32, (_PK * _D_OUT, _LW), 1)
        w8 = jnp.where(rw // _D_OUT == cw // _N_LABELS, wt, 0.0)
        cos = jnp.dot(xn, w8, preferred_element_type=jnp.float32)

        sine = jnp.sqrt(jnp.clip(1.0 - cos * cos, 0.0, 1.0))
        phi = cos * _COS_M - sine * _SIN_M
        phi = jnp.where(cos > _TH, phi, cos - _MM)

        cls = lax.broadcasted_iota(jnp.int32, (rows, _LW), 1) % _N_LABELS
        rs = lax.broadcasted_iota(jnp.int32, (_PK, _LW), 0)
        cs = lax.broadcasted_iota(jnp.int32, (_PK, _LW), 1)
        s32 = jnp.where(rs == cs // _N_LABELS, 1.0, 0.0)
        labelb = jnp.dot(l_ref[...], s32, preferred_element_type=jnp.float32)
        onehot = cls.astype(jnp.float32) == labelb
        out_ref[...] = jnp.where(onehot, phi, cos) * _S

    return pl.pallas_call(
        body,
        grid=grid,
        in_specs=[
            pl.BlockSpec((_NC, rows, _PK * _D_OUT), lambda i: (0, i, 0)),
            pl.BlockSpec((_NC, rows, _PK * _D_OUT), lambda i: (0, i, 0)),
            pl.BlockSpec((rows, _PK * _D_OUT), lambda i: (i, 0)),
            pl.BlockSpec((rows, _PK), lambda i: (i, 0)),
            pl.BlockSpec((_N_LABELS, _D_OUT), lambda i: (0, 0)),
        ],
        out_specs=[
            pl.BlockSpec((rows, _PK * _D_OUT), lambda i: (i, 0)),
            pl.BlockSpec((rows, _LW), lambda i: (i, 0)),
        ],
        out_shape=[
            jax.ShapeDtypeStruct((_N // _PK, _PK * _D_OUT), jnp.float32),
            jax.ShapeDtypeStruct((_N // _PK, _LW), jnp.float32),
        ],
    )(agg_p, cnt_p, z_p, label_p, weight)


def kernel(x, edge_index, label, Wl, bl, Wr, br, weight):
    y, z = _project(x, Wl, Wr, bl.reshape(1, _D_OUT), br.reshape(1, _D_OUT))
    ei = edge_index.reshape(2, _NW, _K, _C)
    agg, cnt = _segment_sum_sc(y, ei)
    np = _N // 8
    feat_p, out_p = _head(
        agg.reshape(_NC, np, 128), cnt.reshape(_NC, _NP // 8, 128),
        z.reshape(np, 128), label.astype(jnp.float32).reshape(np, 8), weight)
    return (feat_p.reshape(_N, _D_OUT), out_p.reshape(_N, _N_LABELS))
